# baseline (speedup 1.0000x reference)
import functools

import jax
import jax.numpy as jnp
from jax import lax
from jax.experimental import pallas as pl
from jax.experimental.pallas import tpu as pltpu
from jax.experimental.pallas import tpu_sc as plsc

_TEMPLATE_SIZE = 100000
_TEMPLATE_FACTOR = 1000
_EMBED_DIM = 64
_BATCH = 16384
_MAX_NORM = 1.0
_SCALE = _TEMPLATE_SIZE / _TEMPLATE_FACTOR

_NC = 2
_NS = 16
_NW = _NC * _NS
_TOTAL_ROWS = _BATCH * 2
_ROWS_PER_W = _TOTAL_ROWS // _NW
_GROUP = 128
_NGROUPS = _ROWS_PER_W // _GROUP


def _tc_prep(anno_ref, w_ref, pos_ref, idx_ref, table_ref):
    anno = anno_ref[...]
    w = anno[:, 2]
    h = anno[:, 3]
    tw = (_SCALE * jnp.sqrt(w / h)).astype(jnp.int32)
    th = (_SCALE * jnp.sqrt(h / w)).astype(jnp.int32)
    tw = jnp.clip(tw, 0, _TEMPLATE_FACTOR - 1)
    th = jnp.clip(th, 0, _TEMPLATE_FACTOR - 1) + _TEMPLATE_FACTOR
    idx_ref[...] = jnp.stack([tw, th], axis=1)

    wt = w_ref[...]
    norm = jnp.sqrt(jnp.sum(wt * wt, axis=1, keepdims=True))
    scale = jnp.where(norm > _MAX_NORM, _MAX_NORM / (norm + 1e-7),
                      jnp.ones_like(norm))
    scaled = wt * scale
    pos = pos_ref[...]
    table_ref[0:_TEMPLATE_FACTOR, :] = scaled + pos[0, 0, :][None, :]
    table_ref[_TEMPLATE_FACTOR:2 * _TEMPLATE_FACTOR, :] = (
        scaled + pos[0, 1, :][None, :])


_tc_prep_call = pl.pallas_call(
    _tc_prep,
    out_shape=(
        jax.ShapeDtypeStruct((_BATCH, 2), jnp.int32),
        jax.ShapeDtypeStruct((2 * _TEMPLATE_FACTOR, _EMBED_DIM), jnp.float32),
    ),
)


@functools.partial(
    pl.kernel,
    mesh=plsc.VectorSubcoreMesh(core_axis_name="c", subcore_axis_name="s"),
    out_type=jax.ShapeDtypeStruct((_TOTAL_ROWS, _EMBED_DIM), jnp.float32),
    scratch_types=[
        pltpu.VMEM((_NGROUPS, _GROUP), jnp.int32),
        pltpu.VMEM((_ROWS_PER_W, _EMBED_DIM), jnp.float32),
        pltpu.SemaphoreType.DMA,
    ],
    compiler_params=pltpu.CompilerParams(use_tc_tiling_on_sc=False),
)
def _sc_gather(table_hbm, idx_hbm, out_hbm, idx_v, rows_v, sem):
    wid = lax.axis_index("s") * _NC + lax.axis_index("c")
    pltpu.sync_copy(idx_hbm.at[wid], idx_v)
    copies = []
    for g in range(_NGROUPS):
        copies.append(
            pltpu.async_copy(
                table_hbm.at[idx_v.at[g]],
                rows_v.at[pl.ds(g * _GROUP, _GROUP)],
                sem,
            ))
    for c in copies:
        c.wait()
    pltpu.sync_copy(rows_v, out_hbm.at[pl.ds(wid * _ROWS_PER_W, _ROWS_PER_W)])


def kernel(template_anno, weight, pos_embed):
    idx, table = _tc_prep_call(template_anno, weight[:_TEMPLATE_FACTOR],
                               pos_embed)
    out_flat = _sc_gather(table, idx.reshape(_NW, _NGROUPS, _GROUP))
    return out_flat.reshape(_BATCH, 2, _EMBED_DIM)
